# baseline (speedup 1.0000x reference)
import functools

import numpy as np
import jax
import jax.numpy as jnp
from jax import lax
from jax.experimental import pallas as pl
from jax.experimental.pallas import tpu as pltpu
from jax.experimental.pallas import tpu_sc as plsc

D_MODEL = 768
MAX_LEN = 8192
NUM_CORES = 2
NUM_SUBCORES = 16
NUM_TILES = NUM_CORES * NUM_SUBCORES
CHUNK = 16
N_SEG = 1


def _pos_encoding(max_len, d_model):
    pos = np.arange(max_len, dtype=np.float32)[:, None]
    i = np.arange(0, d_model, 2, dtype=np.float32)
    div = np.power(10000.0, i / d_model)
    enc = np.zeros((max_len, d_model), dtype=np.float32)
    enc[:, 0::2] = np.sin(pos / div)
    enc[:, 1::2] = np.cos(pos / div)
    return enc


_POS_ENC_NP = _pos_encoding(MAX_LEN, D_MODEL)


N_BUF = 8
AHEAD = 4


def _sc_gather(idx_seg, table, n_rows, d):
    rows_per_tile = n_rows // NUM_TILES
    n_chunks = rows_per_tile // CHUNK

    mesh = plsc.VectorSubcoreMesh(core_axis_name="c", subcore_axis_name="s")

    @functools.partial(
        pl.kernel,
        out_type=jax.ShapeDtypeStruct((n_rows, d), jnp.float32),
        mesh=mesh,
        scratch_types=(
            [pltpu.VMEM((rows_per_tile,), jnp.int32)]
            + [pltpu.VMEM((CHUNK, d), jnp.float32) for _ in range(N_BUF)]
            + [pltpu.SemaphoreType.DMA] * (1 + 2 * N_BUF)
        ),
    )
    def gather_kernel(idx_hbm, table_hbm, out_hbm, idx_v, *rest):
        gbufs = rest[:N_BUF]
        isem = rest[N_BUF]
        gsems = rest[N_BUF + 1:N_BUF + 1 + N_BUF]
        wsems = rest[N_BUF + 1 + N_BUF:]

        wid = lax.axis_index("c") * NUM_SUBCORES + lax.axis_index("s")
        base = wid * rows_per_tile
        idx_cp = pltpu.make_async_copy(
            idx_hbm.at[pl.ds(base, rows_per_tile)], idx_v, isem
        )
        idx_cp.start()
        idx_cp.wait()

        def gather(t):
            b = t % N_BUF
            idx_vec = idx_v[pl.ds(t * CHUNK, CHUNK)]
            return pltpu.make_async_copy(
                table_hbm.at[idx_vec],
                gbufs[b], gsems[b],
            )

        def writeback(t):
            b = t % N_BUF
            return pltpu.make_async_copy(
                gbufs[b], out_hbm.at[pl.ds(base + t * CHUNK, CHUNK)],
                wsems[b],
            )

        for t in range(AHEAD):
            gather(t).start()
        for t in range(n_chunks):
            gather(t).wait()
            writeback(t).start()
            if t >= N_BUF - AHEAD:
                writeback(t - (N_BUF - AHEAD)).wait()
            if t + AHEAD < n_chunks:
                gather(t + AHEAD).start()
        for t in range(n_chunks - (N_BUF - AHEAD), n_chunks):
            writeback(t).wait()

    return gather_kernel(idx_seg, table)


def _tc_add_pos(tok_seg, pos_enc, n_rows, seq_len, d):
    block_rows = 512
    pos_blocks = seq_len // block_rows

    def add_kernel(tok_ref, pos_ref, out_ref):
        out_ref[...] = tok_ref[...] + pos_ref[...]

    return pl.pallas_call(
        add_kernel,
        out_shape=jax.ShapeDtypeStruct((n_rows, d), jnp.float32),
        grid=(n_rows // block_rows,),
        in_specs=[
            pl.BlockSpec((block_rows, d), lambda i: (i, 0)),
            pl.BlockSpec((block_rows, d), lambda i: (i % pos_blocks, 0)),
        ],
        out_specs=pl.BlockSpec((block_rows, d), lambda i: (i, 0)),
    )(tok_seg, pos_enc)


def kernel(x, table):
    batch, seq_len = x.shape
    d = table.shape[1]
    pos_enc = jnp.asarray(_POS_ENC_NP[:seq_len])

    idx_flat = x.reshape(batch * seq_len)
    seg_rows = batch * seq_len // N_SEG

    toks = []
    for s in range(N_SEG):
        idx_seg = lax.slice(idx_flat, (s * seg_rows,), ((s + 1) * seg_rows,))
        toks.append(_sc_gather(idx_seg, table, seg_rows, d))
    outs = [_tc_add_pos(tok, pos_enc, seg_rows, seq_len, d) for tok in toks]
    out = jnp.concatenate(outs, axis=0)
    return out.reshape(batch, seq_len, d)


# speedup vs baseline: 1.0311x; 1.0311x over previous
import functools

import numpy as np
import jax
import jax.numpy as jnp
from jax import lax
from jax.experimental import pallas as pl
from jax.experimental.pallas import tpu as pltpu
from jax.experimental.pallas import tpu_sc as plsc

D_MODEL = 768
MAX_LEN = 8192
NUM_CORES = 2
NUM_SUBCORES = 16
NUM_TILES = NUM_CORES * NUM_SUBCORES
CHUNK = 16
N_SEG = 1


def _pos_encoding(max_len, d_model):
    pos = np.arange(max_len, dtype=np.float32)[:, None]
    i = np.arange(0, d_model, 2, dtype=np.float32)
    div = np.power(10000.0, i / d_model)
    enc = np.zeros((max_len, d_model), dtype=np.float32)
    enc[:, 0::2] = np.sin(pos / div)
    enc[:, 1::2] = np.cos(pos / div)
    return enc


_POS_ENC_NP = _pos_encoding(MAX_LEN, D_MODEL)


N_BUF = 8
AHEAD = 4


def _sc_gather(idx_seg, table, n_rows, d):
    rows_per_tile = n_rows // NUM_TILES
    n_chunks = rows_per_tile // CHUNK

    mesh = plsc.VectorSubcoreMesh(core_axis_name="c", subcore_axis_name="s")

    @functools.partial(
        pl.kernel,
        out_type=jax.ShapeDtypeStruct((n_rows, d), jnp.float32),
        mesh=mesh,
        scratch_types=(
            [pltpu.VMEM((rows_per_tile,), jnp.int32)]
            + [pltpu.VMEM((CHUNK, d), jnp.float32) for _ in range(N_BUF)]
            + [pltpu.SemaphoreType.DMA] * (1 + 2 * N_BUF)
        ),
    )
    def gather_kernel(idx_hbm, table_hbm, out_hbm, idx_v, *rest):
        gbufs = rest[:N_BUF]
        isem = rest[N_BUF]
        gsems = rest[N_BUF + 1:N_BUF + 1 + N_BUF]
        wsems = rest[N_BUF + 1 + N_BUF:]

        wid = lax.axis_index("c") * NUM_SUBCORES + lax.axis_index("s")
        base = wid * rows_per_tile
        idx_cp = pltpu.make_async_copy(
            idx_hbm.at[pl.ds(base, rows_per_tile)], idx_v, isem
        )
        idx_cp.start()
        idx_cp.wait()

        def gather(t):
            b = t % N_BUF
            idx_vec = idx_v[pl.ds(t * CHUNK, CHUNK)]
            return pltpu.make_async_copy(
                table_hbm.at[idx_vec],
                gbufs[b], gsems[b],
            )

        def writeback(t):
            b = t % N_BUF
            return pltpu.make_async_copy(
                gbufs[b], out_hbm.at[pl.ds(base + t * CHUNK, CHUNK)],
                wsems[b],
            )

        for t in range(AHEAD):
            gather(t).start()
        for t in range(n_chunks):
            gather(t).wait()
            writeback(t).start()
            if t >= N_BUF - AHEAD:
                writeback(t - (N_BUF - AHEAD)).wait()
            if t + AHEAD < n_chunks:
                gather(t + AHEAD).start()
        for t in range(n_chunks - (N_BUF - AHEAD), n_chunks):
            writeback(t).wait()

    return gather_kernel(idx_seg, table)


def _tc_add_pos(tok_seg, pos_enc, n_rows, seq_len, d):
    block_rows = 1024
    pos_blocks = seq_len // block_rows

    def add_kernel(tok_ref, pos_ref, out_ref):
        out_ref[...] = tok_ref[...] + pos_ref[...]

    return pl.pallas_call(
        add_kernel,
        out_shape=jax.ShapeDtypeStruct((n_rows, d), jnp.float32),
        grid=(n_rows // block_rows,),
        in_specs=[
            pl.BlockSpec((block_rows, d), lambda i: (i, 0)),
            pl.BlockSpec((block_rows, d), lambda i: (i % pos_blocks, 0)),
        ],
        out_specs=pl.BlockSpec((block_rows, d), lambda i: (i, 0)),
        compiler_params=pltpu.CompilerParams(
            dimension_semantics=("parallel",),
        ),
    )(tok_seg, pos_enc)


def kernel(x, table):
    batch, seq_len = x.shape
    d = table.shape[1]
    pos_enc = jnp.asarray(_POS_ENC_NP[:seq_len])

    idx_flat = x.reshape(batch * seq_len)
    seg_rows = batch * seq_len // N_SEG

    toks = []
    for s in range(N_SEG):
        idx_seg = lax.slice(idx_flat, (s * seg_rows,), ((s + 1) * seg_rows,))
        toks.append(_sc_gather(idx_seg, table, seg_rows, d))
    outs = [_tc_add_pos(tok, pos_enc, seg_rows, seq_len, d) for tok in toks]
    out = jnp.concatenate(outs, axis=0)
    return out.reshape(batch, seq_len, d)


# speedup vs baseline: 1.1360x; 1.1018x over previous
import functools

import numpy as np
import jax
import jax.numpy as jnp
from jax import lax
from jax.experimental import pallas as pl
from jax.experimental.pallas import tpu as pltpu
from jax.experimental.pallas import tpu_sc as plsc

D_MODEL = 768
MAX_LEN = 8192
NUM_CORES = 2
NUM_SUBCORES = 16
NUM_TILES = NUM_CORES * NUM_SUBCORES
CHUNK = 16
N_SEG = 1


def _pos_encoding(max_len, d_model):
    pos = np.arange(max_len, dtype=np.float32)[:, None]
    i = np.arange(0, d_model, 2, dtype=np.float32)
    div = np.power(10000.0, i / d_model)
    enc = np.zeros((max_len, d_model), dtype=np.float32)
    enc[:, 0::2] = np.sin(pos / div)
    enc[:, 1::2] = np.cos(pos / div)
    return enc


_POS_ENC_NP = _pos_encoding(MAX_LEN, D_MODEL)


N_BUF = 8
AHEAD = 4


def _sc_gather(idx_seg, table, n_rows, d):
    rows_per_tile = n_rows // NUM_TILES
    n_chunks = rows_per_tile // CHUNK

    mesh = plsc.VectorSubcoreMesh(core_axis_name="c", subcore_axis_name="s")

    @functools.partial(
        pl.kernel,
        out_type=jax.ShapeDtypeStruct((n_rows, d), jnp.float32),
        mesh=mesh,
        scratch_types=(
            [pltpu.VMEM((rows_per_tile,), jnp.int32)]
            + [pltpu.VMEM((CHUNK, d), jnp.float32) for _ in range(N_BUF)]
            + [pltpu.SemaphoreType.DMA] * (1 + 2 * N_BUF)
        ),
    )
    def gather_kernel(idx_hbm, table_hbm, out_hbm, idx_v, *rest):
        gbufs = rest[:N_BUF]
        isem = rest[N_BUF]
        gsems = rest[N_BUF + 1:N_BUF + 1 + N_BUF]
        wsems = rest[N_BUF + 1 + N_BUF:]

        wid = lax.axis_index("c") * NUM_SUBCORES + lax.axis_index("s")
        base = wid * rows_per_tile
        idx_cp = pltpu.make_async_copy(
            idx_hbm.at[pl.ds(base, rows_per_tile)], idx_v, isem
        )
        idx_cp.start()
        idx_cp.wait()

        def gather(t):
            b = t % N_BUF
            idx_vec = idx_v[pl.ds(t * CHUNK, CHUNK)]
            return pltpu.make_async_copy(
                table_hbm.at[idx_vec],
                gbufs[b], gsems[b],
            )

        def writeback(t):
            b = t % N_BUF
            return pltpu.make_async_copy(
                gbufs[b], out_hbm.at[pl.ds(base + t * CHUNK, CHUNK)],
                wsems[b],
            )

        for t in range(AHEAD):
            gather(t).start()
        for t in range(n_chunks):
            gather(t).wait()
            writeback(t).start()
            if t >= N_BUF - AHEAD:
                writeback(t - (N_BUF - AHEAD)).wait()
            if t + AHEAD < n_chunks:
                gather(t + AHEAD).start()
        for t in range(n_chunks - (N_BUF - AHEAD), n_chunks):
            writeback(t).wait()

    return gather_kernel(idx_seg, table)


def _tc_add_pos(tok_seg, pos_enc, n_rows, seq_len, d):
    block_rows = 1024
    pos_blocks = seq_len // block_rows

    def add_kernel(tok_ref, pos_ref, out_ref):
        i = pl.program_id(0) % pos_blocks
        out_ref[...] = tok_ref[...] + pos_ref[pl.ds(i * block_rows,
                                                    block_rows), :]

    return pl.pallas_call(
        add_kernel,
        out_shape=jax.ShapeDtypeStruct((n_rows, d), jnp.float32),
        grid=(n_rows // block_rows,),
        in_specs=[
            pl.BlockSpec((block_rows, d), lambda i: (i, 0)),
            pl.BlockSpec((seq_len, d), lambda i: (0, 0)),
        ],
        out_specs=pl.BlockSpec((block_rows, d), lambda i: (i, 0)),
        compiler_params=pltpu.CompilerParams(
            dimension_semantics=("parallel",),
        ),
    )(tok_seg, pos_enc)


def kernel(x, table):
    batch, seq_len = x.shape
    d = table.shape[1]
    pos_enc = jnp.asarray(_POS_ENC_NP[:seq_len])

    idx_flat = x.reshape(batch * seq_len)
    seg_rows = batch * seq_len // N_SEG

    toks = []
    for s in range(N_SEG):
        idx_seg = lax.slice(idx_flat, (s * seg_rows,), ((s + 1) * seg_rows,))
        toks.append(_sc_gather(idx_seg, table, seg_rows, d))
    outs = [_tc_add_pos(tok, pos_enc, seg_rows, seq_len, d) for tok in toks]
    out = jnp.concatenate(outs, axis=0)
    return out.reshape(batch, seq_len, d)


# speedup vs baseline: 1.1476x; 1.0101x over previous
import functools

import numpy as np
import jax
import jax.numpy as jnp
from jax import lax
from jax.experimental import pallas as pl
from jax.experimental.pallas import tpu as pltpu
from jax.experimental.pallas import tpu_sc as plsc

D_MODEL = 768
MAX_LEN = 8192
NUM_CORES = 2
NUM_SUBCORES = 16
NUM_TILES = NUM_CORES * NUM_SUBCORES
CHUNK = 16
N_SEG = 1


def _pos_encoding(max_len, d_model):
    pos = np.arange(max_len, dtype=np.float32)[:, None]
    i = np.arange(0, d_model, 2, dtype=np.float32)
    div = np.power(10000.0, i / d_model)
    enc = np.zeros((max_len, d_model), dtype=np.float32)
    enc[:, 0::2] = np.sin(pos / div)
    enc[:, 1::2] = np.cos(pos / div)
    return enc


_POS_ENC_NP = _pos_encoding(MAX_LEN, D_MODEL)


N_BUF = 8
AHEAD = 4


def _sc_gather(idx_seg, table, n_rows, d):
    rows_per_tile = n_rows // NUM_TILES
    n_chunks = rows_per_tile // CHUNK

    mesh = plsc.VectorSubcoreMesh(core_axis_name="c", subcore_axis_name="s")

    @functools.partial(
        pl.kernel,
        out_type=jax.ShapeDtypeStruct((n_rows, d), jnp.float32),
        mesh=mesh,
        scratch_types=(
            [pltpu.VMEM((rows_per_tile,), jnp.int32)]
            + [pltpu.VMEM((CHUNK, d), jnp.float32) for _ in range(N_BUF)]
            + [pltpu.SemaphoreType.DMA] * (1 + 2 * N_BUF)
        ),
    )
    def gather_kernel(idx_hbm, table_hbm, out_hbm, idx_v, *rest):
        gbufs = rest[:N_BUF]
        isem = rest[N_BUF]
        gsems = rest[N_BUF + 1:N_BUF + 1 + N_BUF]
        wsems = rest[N_BUF + 1 + N_BUF:]

        wid = lax.axis_index("c") * NUM_SUBCORES + lax.axis_index("s")
        base = wid * rows_per_tile
        idx_cp = pltpu.make_async_copy(
            idx_hbm.at[pl.ds(base, rows_per_tile)], idx_v, isem
        )
        idx_cp.start()
        idx_cp.wait()

        def gather(t):
            b = t % N_BUF
            idx_vec = idx_v[pl.ds(t * CHUNK, CHUNK)]
            return pltpu.make_async_copy(
                table_hbm.at[idx_vec],
                gbufs[b], gsems[b],
            )

        def writeback(t):
            b = t % N_BUF
            return pltpu.make_async_copy(
                gbufs[b], out_hbm.at[pl.ds(base + t * CHUNK, CHUNK)],
                wsems[b],
            )

        for t in range(AHEAD):
            gather(t).start()
        for t in range(n_chunks):
            gather(t).wait()
            writeback(t).start()
            if t >= N_BUF - AHEAD:
                writeback(t - (N_BUF - AHEAD)).wait()
            if t + AHEAD < n_chunks:
                gather(t + AHEAD).start()
        for t in range(n_chunks - (N_BUF - AHEAD), n_chunks):
            writeback(t).wait()

    return gather_kernel(idx_seg, table)


def _tc_add_pos(tok_seg, pos_enc, n_rows, seq_len, d):
    block_rows = 2048
    pos_blocks = seq_len // block_rows

    def add_kernel(tok_ref, pos_ref, out_ref):
        i = pl.program_id(0) % pos_blocks
        out_ref[...] = tok_ref[...] + pos_ref[pl.ds(i * block_rows,
                                                    block_rows), :]

    return pl.pallas_call(
        add_kernel,
        out_shape=jax.ShapeDtypeStruct((n_rows, d), jnp.float32),
        grid=(n_rows // block_rows,),
        in_specs=[
            pl.BlockSpec((block_rows, d), lambda i: (i, 0)),
            pl.BlockSpec((seq_len, d), lambda i: (0, 0)),
        ],
        out_specs=pl.BlockSpec((block_rows, d), lambda i: (i, 0)),
        compiler_params=pltpu.CompilerParams(
            dimension_semantics=("parallel",),
        ),
    )(tok_seg, pos_enc)


def kernel(x, table):
    batch, seq_len = x.shape
    d = table.shape[1]
    pos_enc = jnp.asarray(_POS_ENC_NP[:seq_len])

    idx_flat = x.reshape(batch * seq_len)
    seg_rows = batch * seq_len // N_SEG

    toks = []
    for s in range(N_SEG):
        idx_seg = lax.slice(idx_flat, (s * seg_rows,), ((s + 1) * seg_rows,))
        toks.append(_sc_gather(idx_seg, table, seg_rows, d))
    outs = [_tc_add_pos(tok, pos_enc, seg_rows, seq_len, d) for tok in toks]
    out = jnp.concatenate(outs, axis=0)
    return out.reshape(batch, seq_len, d)
